# baseline (speedup 1.0000x reference)
import jax
import jax.numpy as jnp
from jax import lax
from jax.experimental import pallas as pl
from jax.experimental.pallas import tpu as pltpu
from jax.experimental.pallas import tpu_sc as plsc

N = 10000
D = 128
E = 320000

NC = 2
NS = 16
NW = NC * NS
B = 128
GSZ = 8
NBLK = 80
QG = NBLK // GSZ
EPW = NBLK * B
EPAD = EPW * NW
NPAD = 112
NACC = N + NPAD
RPS = NACC // NS
RB = 128
GRID = NACC // RB

_mesh = plsc.VectorSubcoreMesh(core_axis_name="c", subcore_axis_name="s")


def _sc_deg_body(dstp8_hbm, out_hbm, didx8, ones_v, zbuf, dacc, isem, ssem):
    c = lax.axis_index("c")
    s = lax.axis_index("s")
    wid = c * NS + s
    gbase = wid * QG
    for j in range(B // 16):
        ones_v[pl.ds(j * 16, 16)] = jnp.ones((16,), jnp.float32)

    def zb(i, carry):
        zbuf[pl.ds(i * 16, 16)] = jnp.zeros((16,), jnp.float32)
        return carry

    lax.fori_loop(0, RPS // 16, zb, 0)
    zbuf[pl.ds(RPS - 16, 16)] = jnp.zeros((16,), jnp.float32)
    pltpu.sync_copy(zbuf, dacc.at[pl.ds(s * RPS, RPS)])

    def issue_idx(t):
        tb = lax.rem(t, 4)
        pltpu.async_copy(dstp8_hbm.at[gbase + t], didx8.at[tb], isem.at[tb])

    def wait_idx(t):
        tb = lax.rem(t, 4)
        pltpu.make_async_copy(dstp8_hbm.at[gbase], didx8.at[tb], isem.at[tb]).wait()

    def start_scatter(qb, u):
        pltpu.async_copy(ones_v, dacc.at[didx8.at[qb, u]],
                         ssem.at[u % 2], add=True)

    def wait_scatter(b):
        pltpu.make_async_copy(ones_v, dacc.at[didx8.at[0, 0]],
                              ssem.at[b]).wait()

    issue_idx(0)
    issue_idx(1)
    issue_idx(2)
    plsc.subcore_barrier()
    wait_idx(0)

    def body(q, carry):
        qb = lax.rem(q, 4)
        for u in range(GSZ):
            i = q * GSZ + u
            if u == 2:
                @pl.when(q + 3 < QG)
                def _():
                    issue_idx(q + 3)
            if u == 5:
                @pl.when(q + 1 < QG)
                def _():
                    wait_idx(q + 1)

            @pl.when(i >= 2)
            def _():
                wait_scatter(u % 2)

            start_scatter(qb, u)
        return carry

    lax.fori_loop(0, QG, body, 0)
    wait_scatter(0)
    wait_scatter(1)
    plsc.subcore_barrier()
    pltpu.sync_copy(dacc.at[pl.ds(s * RPS, RPS)], zbuf)
    pltpu.sync_copy(zbuf, out_hbm.at[pl.ds(c * NACC + s * RPS, RPS)])


def _sc_deg(dstp8):
    return pl.kernel(
        _sc_deg_body,
        out_type=jax.ShapeDtypeStruct((NC * NACC,), jnp.float32),
        mesh=_mesh,
        scratch_types=[
            pltpu.VMEM((4, GSZ, B), jnp.int32),
            pltpu.VMEM((B,), jnp.float32),
            pltpu.VMEM((RPS,), jnp.float32),
            pltpu.VMEM_SHARED((NACC,), jnp.float32),
            pltpu.SemaphoreType.DMA((4,)),
            pltpu.SemaphoreType.DMA((2,)),
        ],
    )(dstp8)


def _sc_agg_body(tbl_hbm, srcp8_hbm, dstp8_hbm, zeros_hbm, out_hbm,
                 sidx8, didx8, rows, acc, isem, gsem, ssem):
    c = lax.axis_index("c")
    s = lax.axis_index("s")
    wid = c * NS + s
    gbase = wid * QG
    pltpu.sync_copy(zeros_hbm, acc.at[pl.ds(s * RPS, RPS)])

    def issue_idx(t):
        tb = lax.rem(t, 4)
        pltpu.async_copy(srcp8_hbm.at[gbase + t], sidx8.at[tb], isem.at[tb])
        pltpu.async_copy(dstp8_hbm.at[gbase + t], didx8.at[tb], isem.at[tb])

    def wait_idx(t):
        tb = lax.rem(t, 4)
        pltpu.make_async_copy(srcp8_hbm.at[gbase], sidx8.at[tb], isem.at[tb]).wait()
        pltpu.make_async_copy(dstp8_hbm.at[gbase], didx8.at[tb], isem.at[tb]).wait()

    def start_gather(qg, ug):
        b = ug % 2
        pltpu.async_copy(tbl_hbm.at[sidx8.at[qg, ug]], rows.at[b], gsem.at[b])

    def wait_gather(b):
        pltpu.make_async_copy(tbl_hbm.at[sidx8.at[0, 0]], rows.at[b],
                              gsem.at[b]).wait()

    def start_scatter(qb, u):
        pltpu.async_copy(rows.at[u % 2], acc.at[didx8.at[qb, u]],
                         ssem.at[u % 2], add=True)

    def wait_scatter(b):
        pltpu.make_async_copy(rows.at[b], acc.at[didx8.at[0, 0]],
                              ssem.at[b]).wait()

    issue_idx(0)
    issue_idx(1)
    issue_idx(2)
    plsc.subcore_barrier()
    wait_idx(0)
    start_gather(0, 0)

    def body(q, carry):
        qb = lax.rem(q, 4)
        for u in range(GSZ):
            i = q * GSZ + u
            if u == 2:
                @pl.when(q + 3 < QG)
                def _():
                    issue_idx(q + 3)
            if u == 5:
                @pl.when(q + 1 < QG)
                def _():
                    wait_idx(q + 1)
            wait_gather(u % 2)
            start_scatter(qb, u)

            @pl.when(i >= 1)
            def _():
                wait_scatter((u + 1) % 2)

            @pl.when(i + 1 < NBLK)
            def _():
                k = i + 1
                start_gather(lax.rem(k // GSZ, 4), (u + 1) % GSZ)
        return carry

    lax.fori_loop(0, QG, body, 0)
    wait_scatter((NBLK - 1) % 2)
    plsc.subcore_barrier()
    pltpu.sync_copy(acc.at[pl.ds(s * RPS, RPS)],
                    out_hbm.at[pl.ds(c * NACC + s * RPS, RPS)])


def _sc_agg(tbl, srcp8, dstp8, zerosD):
    return pl.kernel(
        _sc_agg_body,
        out_type=jax.ShapeDtypeStruct((NC * NACC, D), jnp.float32),
        mesh=_mesh,
        scratch_types=[
            pltpu.VMEM((4, GSZ, B), jnp.int32),
            pltpu.VMEM((4, GSZ, B), jnp.int32),
            pltpu.VMEM((2, B, D), jnp.float32),
            pltpu.VMEM_SHARED((NACC, D), jnp.float32),
            pltpu.SemaphoreType.DMA((4,)),
            pltpu.SemaphoreType.DMA((2,)),
            pltpu.SemaphoreType.DMA((2,)),
        ],
    )(tbl, srcp8, dstp8, zerosD)


def _dinv_block(d0_ref, d1_ref):
    d = d0_ref[...] + d1_ref[...] + 1.0
    return lax.rsqrt(d).reshape(RB, 1)


def _tc_scale_body(d0_ref, d1_ref, emb_ref, o_ref):
    i = pl.program_id(0)
    rowg = i * RB + lax.broadcasted_iota(jnp.int32, (RB, 1), 0)
    x = jnp.where(rowg < N, emb_ref[...], 0.0)
    o_ref[...] = x * _dinv_block(d0_ref, d1_ref)


def _tc_mm_body(d0_ref, d1_ref, s1a_ref, s1b_ref, x1_ref, w1_ref, b1_ref,
                w2_ref, o_ref):
    dinv = _dinv_block(d0_ref, d1_ref)
    s1 = (s1a_ref[...] + s1b_ref[...] + x1_ref[...]) * dinv
    h = jnp.dot(s1, w1_ref[...], preferred_element_type=jnp.float32) + b1_ref[...]
    v = jnp.dot(h, w2_ref[...], preferred_element_type=jnp.float32)
    o_ref[...] = v * dinv


def _tc_fin_body(d0_ref, d1_ref, s2a_ref, s2b_ref, vp_ref, b2_ref, g_ref,
                 be_ref, o_ref, st_ref):
    p = pl.program_id(0)
    i = pl.program_id(1)
    dinv = _dinv_block(d0_ref, d1_ref)
    pre = (s2a_ref[...] + s2b_ref[...] + vp_ref[...]) * dinv + b2_ref[...]

    @pl.when(p == 0)
    def _():
        @pl.when(i == 0)
        def _():
            st_ref[...] = jnp.zeros_like(st_ref)

        rowg = i * RB + lax.broadcasted_iota(jnp.int32, (RB, 1), 0)
        prem = jnp.where(rowg < N, pre, 0.0)
        sm = jnp.sum(prem, axis=0, keepdims=True)
        sq = jnp.sum(prem * prem, axis=0, keepdims=True)
        st_ref[...] += jnp.concatenate(
            [sm, sq, jnp.zeros((6, D), jnp.float32)], axis=0)
        o_ref[...] = pre

    @pl.when(p == 1)
    def _():
        mean = st_ref[0:1, :] / N
        ex2 = st_ref[1:2, :] / N
        var = ex2 - mean * mean
        o_ref[...] = (pre - mean) * lax.rsqrt(var + 1e-5) * g_ref[...] + be_ref[...]


_blk = lambda i: (i, 0)
_whole = lambda i: (0, 0)
_d1blk = lambda i: (i + NACC // RB, 0)


def _row_spec():
    return pl.BlockSpec((RB, D), _blk)


def _tc_scale(degf, emb):
    return pl.pallas_call(
        _tc_scale_body,
        grid=(GRID,),
        in_specs=[
            pl.BlockSpec((1, 1, RB), lambda i: (i, 0, 0)),
            pl.BlockSpec((1, 1, RB), lambda i: (i + NACC // RB, 0, 0)),
            _row_spec(),
        ],
        out_specs=_row_spec(),
        out_shape=jax.ShapeDtypeStruct((NACC, D), jnp.float32),
    )(degf, degf, emb)


def _tc_mm(degf, s1, x1p, W1, b1, W2):
    return pl.pallas_call(
        _tc_mm_body,
        grid=(GRID,),
        in_specs=[
            pl.BlockSpec((1, 1, RB), lambda i: (i, 0, 0)),
            pl.BlockSpec((1, 1, RB), lambda i: (i + NACC // RB, 0, 0)),
            pl.BlockSpec((RB, D), _blk),
            pl.BlockSpec((RB, D), lambda i: (i + NACC // RB, 0)),
            _row_spec(),
            pl.BlockSpec((D, 2 * D), _whole),
            pl.BlockSpec((1, 2 * D), _whole),
            pl.BlockSpec((2 * D, D), _whole),
        ],
        out_specs=_row_spec(),
        out_shape=jax.ShapeDtypeStruct((NACC, D), jnp.float32),
    )(degf, degf, s1, s1, x1p, W1, b1, W2)


def _tc_fin(degf, s2, vpp, b2, gamma, beta):
    blk2 = lambda p, i: (i, 0)
    whole2 = lambda p, i: (0, 0)
    d1blk2 = lambda p, i: (i + NACC // RB, 0)
    return pl.pallas_call(
        _tc_fin_body,
        grid=(2, GRID),
        in_specs=[
            pl.BlockSpec((1, 1, RB), lambda p, i: (i, 0, 0)),
            pl.BlockSpec((1, 1, RB), lambda p, i: (i + NACC // RB, 0, 0)),
            pl.BlockSpec((RB, D), blk2),
            pl.BlockSpec((RB, D), lambda p, i: (i + NACC // RB, 0)),
            pl.BlockSpec((RB, D), blk2),
            pl.BlockSpec((1, D), whole2),
            pl.BlockSpec((1, D), whole2),
            pl.BlockSpec((1, D), whole2),
        ],
        out_specs=pl.BlockSpec((RB, D), blk2),
        out_shape=jax.ShapeDtypeStruct((N, D), jnp.float32),
        scratch_shapes=[pltpu.VMEM((8, D), jnp.float32)],
    )(degf, degf, s2, s2, vpp, b2, gamma, beta)


def kernel(edge_index, emb, W1, b1, W2, b2, gamma, beta):
    src = edge_index[0].astype(jnp.int32)
    dst = edge_index[1].astype(jnp.int32)
    pad = (N + jnp.arange(EPAD - E, dtype=jnp.int32) % NPAD)
    srcp = jnp.concatenate([src, pad])
    dstp = jnp.concatenate([dst, pad])
    srcp8 = srcp.reshape(NW * QG, GSZ, B)
    dstp8 = dstp.reshape(NW * QG, GSZ, B)

    zerosD = jnp.zeros((RPS, D), jnp.float32)

    degp_flat = _sc_deg(dstp8)
    degf = degp_flat.reshape(NC * NACC // RB, 1, RB)

    x1p = _tc_scale(degf, emb)

    s1 = _sc_agg(x1p, srcp8, dstp8, zerosD)

    vpp = _tc_mm(degf, s1, x1p, W1,
                 b1.reshape(1, 2 * D), W2)

    s2 = _sc_agg(vpp, srcp8, dstp8, zerosD)

    return _tc_fin(degf, s2, vpp, b2.reshape(1, D),
                   gamma.reshape(1, D), beta.reshape(1, D))


# speedup vs baseline: 1.2542x; 1.2542x over previous
import jax
import jax.numpy as jnp
from jax import lax
from jax.experimental import pallas as pl
from jax.experimental.pallas import tpu as pltpu
from jax.experimental.pallas import tpu_sc as plsc

N = 10000
D = 128
E = 320000

NC = 2
NS = 16
NW = NC * NS
B = 128
GSZ = 8
NBLK = 80
QG = NBLK // GSZ
EPW = NBLK * B
EPAD = EPW * NW
NPAD = 112
NACC = N + NPAD
RPS = NACC // NS
RB = 128
GRID = NACC // RB
RBM = 632
GRIDM = NACC // RBM

_mesh = plsc.VectorSubcoreMesh(core_axis_name="c", subcore_axis_name="s")


def _sc_deg_body(dstp8_hbm, out_hbm, didx8, ones_v, zbuf, dacc, isem, ssem):
    c = lax.axis_index("c")
    s = lax.axis_index("s")
    wid = c * NS + s
    gbase = wid * QG
    for j in range(B // 16):
        ones_v[pl.ds(j * 16, 16)] = jnp.ones((16,), jnp.float32)

    def zb(i, carry):
        zbuf[pl.ds(i * 16, 16)] = jnp.zeros((16,), jnp.float32)
        return carry

    lax.fori_loop(0, RPS // 16, zb, 0)
    zbuf[pl.ds(RPS - 16, 16)] = jnp.zeros((16,), jnp.float32)
    pltpu.sync_copy(zbuf, dacc.at[pl.ds(s * RPS, RPS)])

    def issue_idx(t):
        tb = lax.rem(t, 4)
        pltpu.async_copy(dstp8_hbm.at[gbase + t], didx8.at[tb], isem.at[tb])

    def wait_idx(t):
        tb = lax.rem(t, 4)
        pltpu.make_async_copy(dstp8_hbm.at[gbase], didx8.at[tb], isem.at[tb]).wait()

    def start_scatter(qb, u):
        pltpu.async_copy(ones_v, dacc.at[didx8.at[qb, u]],
                         ssem.at[u % 2], add=True)

    def wait_scatter(b):
        pltpu.make_async_copy(ones_v, dacc.at[didx8.at[0, 0]],
                              ssem.at[b]).wait()

    issue_idx(0)
    issue_idx(1)
    issue_idx(2)
    plsc.subcore_barrier()
    wait_idx(0)

    def body(q, carry):
        qb = lax.rem(q, 4)
        for u in range(GSZ):
            i = q * GSZ + u
            if u == 2:
                @pl.when(q + 3 < QG)
                def _():
                    issue_idx(q + 3)
            if u == 5:
                @pl.when(q + 1 < QG)
                def _():
                    wait_idx(q + 1)

            @pl.when(i >= 2)
            def _():
                wait_scatter(u % 2)

            start_scatter(qb, u)
        return carry

    lax.fori_loop(0, QG, body, 0)
    wait_scatter(0)
    wait_scatter(1)
    plsc.subcore_barrier()
    pltpu.sync_copy(dacc.at[pl.ds(s * RPS, RPS)], zbuf)
    pltpu.sync_copy(zbuf, out_hbm.at[pl.ds(c * NACC + s * RPS, RPS)])


def _sc_deg(dstp8):
    return pl.kernel(
        _sc_deg_body,
        out_type=jax.ShapeDtypeStruct((NC * NACC,), jnp.float32),
        mesh=_mesh,
        scratch_types=[
            pltpu.VMEM((4, GSZ, B), jnp.int32),
            pltpu.VMEM((B,), jnp.float32),
            pltpu.VMEM((RPS,), jnp.float32),
            pltpu.VMEM_SHARED((NACC,), jnp.float32),
            pltpu.SemaphoreType.DMA((4,)),
            pltpu.SemaphoreType.DMA((2,)),
        ],
    )(dstp8)


def _sc_agg_body(tbl_hbm, srcp8_hbm, dstp8_hbm, zeros_hbm, out_hbm,
                 sidx8, didx8, rows, acc, isem, gsem, ssem):
    c = lax.axis_index("c")
    s = lax.axis_index("s")
    wid = c * NS + s
    gbase = wid * QG
    pltpu.sync_copy(zeros_hbm, acc.at[pl.ds(s * RPS, RPS)])

    def issue_idx(t):
        tb = lax.rem(t, 4)
        pltpu.async_copy(srcp8_hbm.at[gbase + t], sidx8.at[tb], isem.at[tb])
        pltpu.async_copy(dstp8_hbm.at[gbase + t], didx8.at[tb], isem.at[tb])

    def wait_idx(t):
        tb = lax.rem(t, 4)
        pltpu.make_async_copy(srcp8_hbm.at[gbase], sidx8.at[tb], isem.at[tb]).wait()
        pltpu.make_async_copy(dstp8_hbm.at[gbase], didx8.at[tb], isem.at[tb]).wait()

    def start_gather(qg, ug):
        b = ug % 2
        pltpu.async_copy(tbl_hbm.at[sidx8.at[qg, ug]], rows.at[b], gsem.at[b])

    def wait_gather(b):
        pltpu.make_async_copy(tbl_hbm.at[sidx8.at[0, 0]], rows.at[b],
                              gsem.at[b]).wait()

    def start_scatter(qb, u):
        pltpu.async_copy(rows.at[u % 2], acc.at[didx8.at[qb, u]],
                         ssem.at[u % 2], add=True)

    def wait_scatter(b):
        pltpu.make_async_copy(rows.at[b], acc.at[didx8.at[0, 0]],
                              ssem.at[b]).wait()

    issue_idx(0)
    issue_idx(1)
    issue_idx(2)
    plsc.subcore_barrier()
    wait_idx(0)
    start_gather(0, 0)

    def body(q, carry):
        qb = lax.rem(q, 4)
        for u in range(GSZ):
            i = q * GSZ + u
            if u == 2:
                @pl.when(q + 3 < QG)
                def _():
                    issue_idx(q + 3)
            if u == 5:
                @pl.when(q + 1 < QG)
                def _():
                    wait_idx(q + 1)
            wait_gather(u % 2)
            start_scatter(qb, u)

            @pl.when(i >= 1)
            def _():
                wait_scatter((u + 1) % 2)

            @pl.when(i + 1 < NBLK)
            def _():
                k = i + 1
                start_gather(lax.rem(k // GSZ, 4), (u + 1) % GSZ)
        return carry

    lax.fori_loop(0, QG, body, 0)
    wait_scatter((NBLK - 1) % 2)
    plsc.subcore_barrier()
    pltpu.sync_copy(acc.at[pl.ds(s * RPS, RPS)],
                    out_hbm.at[pl.ds(c * NACC + s * RPS, RPS)])


def _sc_agg(tbl, srcp8, dstp8, zerosD):
    return pl.kernel(
        _sc_agg_body,
        out_type=jax.ShapeDtypeStruct((NC * NACC, D), jnp.float32),
        mesh=_mesh,
        scratch_types=[
            pltpu.VMEM((4, GSZ, B), jnp.int32),
            pltpu.VMEM((4, GSZ, B), jnp.int32),
            pltpu.VMEM((2, B, D), jnp.float32),
            pltpu.VMEM_SHARED((NACC, D), jnp.float32),
            pltpu.SemaphoreType.DMA((4,)),
            pltpu.SemaphoreType.DMA((2,)),
            pltpu.SemaphoreType.DMA((2,)),
        ],
    )(tbl, srcp8, dstp8, zerosD)


def _dinv_block(d0_ref, d1_ref):
    d = d0_ref[...] + d1_ref[...] + 1.0
    return lax.rsqrt(d).reshape(RB, 1)


def _tc_scale_body(d0_ref, d1_ref, emb_ref, o_ref, dv_ref):
    i = pl.program_id(0)
    rowg = i * RB + lax.broadcasted_iota(jnp.int32, (RB, 1), 0)
    x = jnp.where(rowg < N, emb_ref[...], 0.0)
    dinv = _dinv_block(d0_ref, d1_ref)
    dv_ref[...] = dinv
    o_ref[...] = x * dinv


def _tc_mm_body(dv_ref, s1a_ref, s1b_ref, x1_ref, w1_ref, b1_ref,
                w2_ref, o_ref):
    dinv = dv_ref[...]
    s1 = (s1a_ref[...] + s1b_ref[...] + x1_ref[...]) * dinv
    h = jnp.dot(s1, w1_ref[...], preferred_element_type=jnp.float32) + b1_ref[...]
    v = jnp.dot(h, w2_ref[...], preferred_element_type=jnp.float32)
    o_ref[...] = v * dinv


def _tc_fin_body(dv_ref, s2a_ref, s2b_ref, vp_ref, b2_ref, g_ref,
                 be_ref, o_ref, st_ref):
    p = pl.program_id(0)
    i = pl.program_id(1)
    dinv = dv_ref[...]
    pre = (s2a_ref[...] + s2b_ref[...] + vp_ref[...]) * dinv + b2_ref[...]

    @pl.when(p == 0)
    def _():
        @pl.when(i == 0)
        def _():
            st_ref[...] = jnp.zeros_like(st_ref)

        rowg = i * RBM + lax.broadcasted_iota(jnp.int32, (RBM, 1), 0)
        prem = jnp.where(rowg < N, pre, 0.0)
        sm = jnp.sum(prem, axis=0, keepdims=True)
        sq = jnp.sum(prem * prem, axis=0, keepdims=True)
        st_ref[...] += jnp.concatenate(
            [sm, sq, jnp.zeros((6, D), jnp.float32)], axis=0)
        o_ref[...] = pre

    @pl.when(p == 1)
    def _():
        mean = st_ref[0:1, :] / N
        ex2 = st_ref[1:2, :] / N
        var = ex2 - mean * mean
        o_ref[...] = (pre - mean) * lax.rsqrt(var + 1e-5) * g_ref[...] + be_ref[...]


_blk = lambda i: (i, 0)
_whole = lambda i: (0, 0)
_d1blk = lambda i: (i + NACC // RB, 0)


def _row_spec():
    return pl.BlockSpec((RB, D), _blk)


def _tc_scale(degf, emb):
    return pl.pallas_call(
        _tc_scale_body,
        grid=(GRID,),
        in_specs=[
            pl.BlockSpec((1, 1, RB), lambda i: (i, 0, 0)),
            pl.BlockSpec((1, 1, RB), lambda i: (i + NACC // RB, 0, 0)),
            _row_spec(),
        ],
        out_specs=[_row_spec(), pl.BlockSpec((RB, 1), _blk)],
        out_shape=[
            jax.ShapeDtypeStruct((NACC, D), jnp.float32),
            jax.ShapeDtypeStruct((NACC, 1), jnp.float32),
        ],
    )(degf, degf, emb)


def _tc_mm(dinvc, s1, x1p, W1, b1, W2):
    return pl.pallas_call(
        _tc_mm_body,
        grid=(GRIDM,),
        in_specs=[
            pl.BlockSpec((RBM, 1), _blk),
            pl.BlockSpec((RBM, D), _blk),
            pl.BlockSpec((RBM, D), lambda i: (i + NACC // RBM, 0)),
            pl.BlockSpec((RBM, D), _blk),
            pl.BlockSpec((D, 2 * D), _whole),
            pl.BlockSpec((1, 2 * D), _whole),
            pl.BlockSpec((2 * D, D), _whole),
        ],
        out_specs=pl.BlockSpec((RBM, D), _blk),
        out_shape=jax.ShapeDtypeStruct((NACC, D), jnp.float32),
    )(dinvc, s1, s1, x1p, W1, b1, W2)


def _tc_fin(dinvc, s2, vpp, b2, gamma, beta):
    blk2 = lambda p, i: (i, 0)
    whole2 = lambda p, i: (0, 0)
    return pl.pallas_call(
        _tc_fin_body,
        grid=(2, GRIDM),
        in_specs=[
            pl.BlockSpec((RBM, 1), blk2),
            pl.BlockSpec((RBM, D), blk2),
            pl.BlockSpec((RBM, D), lambda p, i: (i + NACC // RBM, 0)),
            pl.BlockSpec((RBM, D), blk2),
            pl.BlockSpec((1, D), whole2),
            pl.BlockSpec((1, D), whole2),
            pl.BlockSpec((1, D), whole2),
        ],
        out_specs=pl.BlockSpec((RBM, D), blk2),
        out_shape=jax.ShapeDtypeStruct((N, D), jnp.float32),
        scratch_shapes=[pltpu.VMEM((8, D), jnp.float32)],
    )(dinvc, s2, s2, vpp, b2, gamma, beta)


def kernel(edge_index, emb, W1, b1, W2, b2, gamma, beta):
    src = edge_index[0].astype(jnp.int32)
    dst = edge_index[1].astype(jnp.int32)
    pad = (N + jnp.arange(EPAD - E, dtype=jnp.int32) % NPAD)
    srcp = jnp.concatenate([src, pad])
    dstp = jnp.concatenate([dst, pad])
    srcp8 = srcp.reshape(NW * QG, GSZ, B)
    dstp8 = dstp.reshape(NW * QG, GSZ, B)

    zerosD = jnp.zeros((RPS, D), jnp.float32)

    degp_flat = _sc_deg(dstp8)
    degf = degp_flat.reshape(NC * NACC // RB, 1, RB)

    x1p, dinvc = _tc_scale(degf, emb)

    s1 = _sc_agg(x1p, srcp8, dstp8, zerosD)

    vpp = _tc_mm(dinvc, s1, x1p, W1,
                 b1.reshape(1, 2 * D), W2)

    s2 = _sc_agg(vpp, srcp8, dstp8, zerosD)

    return _tc_fin(dinvc, s2, vpp, b2.reshape(1, D),
                   gamma.reshape(1, D), beta.reshape(1, D))


# speedup vs baseline: 1.4122x; 1.1259x over previous
import jax
import jax.numpy as jnp
from jax import lax
from jax.experimental import pallas as pl
from jax.experimental.pallas import tpu as pltpu
from jax.experimental.pallas import tpu_sc as plsc

N = 10000
D = 128
E = 320000

NC = 2
NS = 16
NW = NC * NS
B = 128
GSZ = 8
NBLK = 80
QG = NBLK // GSZ
EPW = NBLK * B
EPAD = EPW * NW
NPAD = 112
NACC = N + NPAD
RPS = NACC // NS
RB = 1264
GRID = NACC // RB

_mesh = plsc.VectorSubcoreMesh(core_axis_name="c", subcore_axis_name="s")

import numpy as _np
_PAD_IDX = N + _np.arange(EPAD - E, dtype=_np.int32) % NPAD


def _sc_deg_body(dstp8_hbm, out_hbm, didx8, ones_v, zbuf, dacc, isem, ssem):
    c = lax.axis_index("c")
    s = lax.axis_index("s")
    wid = c * NS + s
    gbase = wid * QG
    for j in range(B // 16):
        ones_v[pl.ds(j * 16, 16)] = jnp.ones((16,), jnp.float32)

    def zb(i, carry):
        zbuf[pl.ds(i * 16, 16)] = jnp.zeros((16,), jnp.float32)
        return carry

    lax.fori_loop(0, RPS // 16, zb, 0)
    zbuf[pl.ds(RPS - 16, 16)] = jnp.zeros((16,), jnp.float32)
    pltpu.sync_copy(zbuf, dacc.at[pl.ds(s * RPS, RPS)])

    def issue_idx(t):
        tb = lax.rem(t, 4)
        pltpu.async_copy(dstp8_hbm.at[gbase + t], didx8.at[tb], isem.at[tb])

    def wait_idx(t):
        tb = lax.rem(t, 4)
        pltpu.make_async_copy(dstp8_hbm.at[gbase], didx8.at[tb], isem.at[tb]).wait()

    def start_scatter(qb, u):
        pltpu.async_copy(ones_v, dacc.at[didx8.at[qb, u]],
                         ssem.at[u % 2], add=True)

    def wait_scatter(b):
        pltpu.make_async_copy(ones_v, dacc.at[didx8.at[0, 0]],
                              ssem.at[b]).wait()

    issue_idx(0)
    issue_idx(1)
    issue_idx(2)
    plsc.subcore_barrier()
    wait_idx(0)

    def body(q, carry):
        qb = lax.rem(q, 4)
        for u in range(GSZ):
            i = q * GSZ + u
            if u == 2:
                @pl.when(q + 3 < QG)
                def _():
                    issue_idx(q + 3)
            if u == 5:
                @pl.when(q + 1 < QG)
                def _():
                    wait_idx(q + 1)

            @pl.when(i >= 2)
            def _():
                wait_scatter(u % 2)

            start_scatter(qb, u)
        return carry

    lax.fori_loop(0, QG, body, 0)
    wait_scatter(0)
    wait_scatter(1)
    plsc.subcore_barrier()
    pltpu.sync_copy(dacc.at[pl.ds(s * RPS, RPS)], zbuf)
    pltpu.sync_copy(zbuf, out_hbm.at[pl.ds(c * NACC + s * RPS, RPS)])


def _sc_deg(dstp8):
    return pl.kernel(
        _sc_deg_body,
        out_type=jax.ShapeDtypeStruct((NC * NACC,), jnp.float32),
        mesh=_mesh,
        scratch_types=[
            pltpu.VMEM((4, GSZ, B), jnp.int32),
            pltpu.VMEM((B,), jnp.float32),
            pltpu.VMEM((RPS,), jnp.float32),
            pltpu.VMEM_SHARED((NACC,), jnp.float32),
            pltpu.SemaphoreType.DMA((4,)),
            pltpu.SemaphoreType.DMA((2,)),
        ],
    )(dstp8)


def _sc_agg_body(tbl_hbm, srcp8_hbm, dstp8_hbm, zeros_hbm, out_hbm,
                 sidx8, didx8, rows, acc, isem, gsem, ssem):
    c = lax.axis_index("c")
    s = lax.axis_index("s")
    wid = c * NS + s
    gbase = wid * QG
    pltpu.sync_copy(zeros_hbm, acc.at[pl.ds(s * RPS, RPS)])

    def issue_idx(t):
        tb = lax.rem(t, 4)
        pltpu.async_copy(srcp8_hbm.at[gbase + t], sidx8.at[tb], isem.at[tb])
        pltpu.async_copy(dstp8_hbm.at[gbase + t], didx8.at[tb], isem.at[tb])

    def wait_idx(t):
        tb = lax.rem(t, 4)
        pltpu.make_async_copy(srcp8_hbm.at[gbase], sidx8.at[tb], isem.at[tb]).wait()
        pltpu.make_async_copy(dstp8_hbm.at[gbase], didx8.at[tb], isem.at[tb]).wait()

    def start_gather(qg, ug):
        b = ug % 2
        pltpu.async_copy(tbl_hbm.at[sidx8.at[qg, ug]], rows.at[b], gsem.at[b])

    def wait_gather(b):
        pltpu.make_async_copy(tbl_hbm.at[sidx8.at[0, 0]], rows.at[b],
                              gsem.at[b]).wait()

    def start_scatter(qb, u):
        pltpu.async_copy(rows.at[u % 2], acc.at[didx8.at[qb, u]],
                         ssem.at[u % 2], add=True)

    def wait_scatter(b):
        pltpu.make_async_copy(rows.at[b], acc.at[didx8.at[0, 0]],
                              ssem.at[b]).wait()

    issue_idx(0)
    issue_idx(1)
    issue_idx(2)
    plsc.subcore_barrier()
    wait_idx(0)
    start_gather(0, 0)

    def body(q, carry):
        qb = lax.rem(q, 4)
        for u in range(GSZ):
            i = q * GSZ + u
            if u == 2:
                @pl.when(q + 3 < QG)
                def _():
                    issue_idx(q + 3)
            if u == 5:
                @pl.when(q + 1 < QG)
                def _():
                    wait_idx(q + 1)
            wait_gather(u % 2)
            start_scatter(qb, u)

            @pl.when(i >= 1)
            def _():
                wait_scatter((u + 1) % 2)

            @pl.when(i + 1 < NBLK)
            def _():
                k = i + 1
                start_gather(lax.rem(k // GSZ, 4), (u + 1) % GSZ)
        return carry

    lax.fori_loop(0, QG, body, 0)
    wait_scatter((NBLK - 1) % 2)
    plsc.subcore_barrier()
    pltpu.sync_copy(acc.at[pl.ds(s * RPS, RPS)],
                    out_hbm.at[pl.ds(c * NACC + s * RPS, RPS)])


def _sc_agg(tbl, srcp8, dstp8, zerosD):
    return pl.kernel(
        _sc_agg_body,
        out_type=jax.ShapeDtypeStruct((NC * NACC, D), jnp.float32),
        mesh=_mesh,
        scratch_types=[
            pltpu.VMEM((4, GSZ, B), jnp.int32),
            pltpu.VMEM((4, GSZ, B), jnp.int32),
            pltpu.VMEM((2, B, D), jnp.float32),
            pltpu.VMEM_SHARED((NACC, D), jnp.float32),
            pltpu.SemaphoreType.DMA((4,)),
            pltpu.SemaphoreType.DMA((2,)),
            pltpu.SemaphoreType.DMA((2,)),
        ],
    )(tbl, srcp8, dstp8, zerosD)


def _dinv_block(degp_ref):
    d = degp_ref[...]
    return lax.rsqrt(d[:, 0:1] + d[:, 1:2] + 1.0)


def _tc_scale_body(degp_ref, emb_ref, o_ref):
    i = pl.program_id(0)
    rowg = i * RB + lax.broadcasted_iota(jnp.int32, (RB, 1), 0)
    x = jnp.where(rowg < N, emb_ref[...], 0.0)
    o_ref[...] = x * _dinv_block(degp_ref)


def _tc_mm_body(degp_ref, s1a_ref, s1b_ref, x1_ref, w1_ref, b1_ref,
                w2_ref, o_ref):
    dinv = _dinv_block(degp_ref)
    s1 = (s1a_ref[...] + s1b_ref[...] + x1_ref[...]) * dinv
    h = jnp.dot(s1, w1_ref[...], preferred_element_type=jnp.float32) + b1_ref[...]
    v = jnp.dot(h, w2_ref[...], preferred_element_type=jnp.float32)
    o_ref[...] = v * dinv


def _tc_fin_body(degp_ref, s2a_ref, s2b_ref, vp_ref, b2_ref, g_ref,
                 be_ref, o_ref, st_ref):
    p = pl.program_id(0)
    i = pl.program_id(1)
    dinv = _dinv_block(degp_ref)
    pre = (s2a_ref[...] + s2b_ref[...] + vp_ref[...]) * dinv + b2_ref[...]

    @pl.when(p == 0)
    def _():
        @pl.when(i == 0)
        def _():
            st_ref[...] = jnp.zeros_like(st_ref)

        rowg = i * RB + lax.broadcasted_iota(jnp.int32, (RB, 1), 0)
        prem = jnp.where(rowg < N, pre, 0.0)
        sm = jnp.sum(prem, axis=0, keepdims=True)
        sq = jnp.sum(prem * prem, axis=0, keepdims=True)
        st_ref[...] += jnp.concatenate(
            [sm, sq, jnp.zeros((6, D), jnp.float32)], axis=0)
        o_ref[...] = pre

    @pl.when(p == 1)
    def _():
        mean = st_ref[0:1, :] / N
        ex2 = st_ref[1:2, :] / N
        var = ex2 - mean * mean
        o_ref[...] = (pre - mean) * lax.rsqrt(var + 1e-5) * g_ref[...] + be_ref[...]


_blk = lambda i: (i, 0)
_whole = lambda i: (0, 0)
_d1blk = lambda i: (i + NACC // RB, 0)


def _row_spec():
    return pl.BlockSpec((RB, D), _blk)


def _tc_scale(degp, emb):
    return pl.pallas_call(
        _tc_scale_body,
        grid=(GRID,),
        in_specs=[
            pl.BlockSpec((RB, 2), _blk),
            _row_spec(),
        ],
        out_specs=_row_spec(),
        out_shape=jax.ShapeDtypeStruct((NACC, D), jnp.float32),
    )(degp, emb)


def _tc_mm(degp, s1, x1p, W1, b1, W2):
    return pl.pallas_call(
        _tc_mm_body,
        grid=(GRID,),
        in_specs=[
            pl.BlockSpec((RB, 2), _blk),
            pl.BlockSpec((RB, D), _blk),
            pl.BlockSpec((RB, D), lambda i: (i + NACC // RB, 0)),
            pl.BlockSpec((RB, D), _blk),
            pl.BlockSpec((D, 2 * D), _whole),
            pl.BlockSpec((1, 2 * D), _whole),
            pl.BlockSpec((2 * D, D), _whole),
        ],
        out_specs=pl.BlockSpec((RB, D), _blk),
        out_shape=jax.ShapeDtypeStruct((NACC, D), jnp.float32),
    )(degp, s1, s1, x1p, W1, b1, W2)


def _tc_fin(degp, s2, vpp, b2, gamma, beta):
    blk2 = lambda p, i: (i, 0)
    whole2 = lambda p, i: (0, 0)
    return pl.pallas_call(
        _tc_fin_body,
        grid=(2, GRID),
        in_specs=[
            pl.BlockSpec((RB, 2), blk2),
            pl.BlockSpec((RB, D), blk2),
            pl.BlockSpec((RB, D), lambda p, i: (i + NACC // RB, 0)),
            pl.BlockSpec((RB, D), blk2),
            pl.BlockSpec((1, D), whole2),
            pl.BlockSpec((1, D), whole2),
            pl.BlockSpec((1, D), whole2),
        ],
        out_specs=pl.BlockSpec((RB, D), blk2),
        out_shape=jax.ShapeDtypeStruct((N, D), jnp.float32),
        scratch_shapes=[pltpu.VMEM((8, D), jnp.float32)],
    )(degp, s2, s2, vpp, b2, gamma, beta)


def kernel(edge_index, emb, W1, b1, W2, b2, gamma, beta):
    src = edge_index[0].astype(jnp.int32)
    dst = edge_index[1].astype(jnp.int32)
    srcp = jnp.concatenate([src, _PAD_IDX])
    dstp = jnp.concatenate([dst, _PAD_IDX])
    srcp8 = srcp.reshape(NW * QG, GSZ, B)
    dstp8 = dstp.reshape(NW * QG, GSZ, B)

    zerosD = jnp.zeros((RPS, D), jnp.float32)

    degp_flat = _sc_deg(dstp8)
    degp = degp_flat.reshape(NC, NACC).T

    x1p = _tc_scale(degp, emb)

    s1 = _sc_agg(x1p, srcp8, dstp8, zerosD)

    vpp = _tc_mm(degp, s1, x1p, W1,
                 b1.reshape(1, 2 * D), W2)

    s2 = _sc_agg(vpp, srcp8, dstp8, zerosD)

    return _tc_fin(degp, s2, vpp, b2.reshape(1, D),
                   gamma.reshape(1, D), beta.reshape(1, D))


# speedup vs baseline: 1.4202x; 1.0057x over previous
import jax
import jax.numpy as jnp
from jax import lax
from jax.experimental import pallas as pl
from jax.experimental.pallas import tpu as pltpu
from jax.experimental.pallas import tpu_sc as plsc

N = 10000
D = 128
E = 320000

NC = 2
NS = 16
NW = NC * NS
B = 128
GSZ = 8
NBLK = 80
QG = NBLK // GSZ
EPW = NBLK * B
EPAD = EPW * NW
NPAD = 112
NACC = N + NPAD
RPS = NACC // NS
RB = 1264
GRID = NACC // RB

_mesh = plsc.VectorSubcoreMesh(core_axis_name="c", subcore_axis_name="s")

import numpy as _np
_PAD_IDX = N + _np.arange(EPAD - E, dtype=_np.int32) % NPAD


def _sc_deg_body(dstp8_hbm, out_hbm, didx8, ones_v, zbuf, dacc, isem, ssem):
    c = lax.axis_index("c")
    s = lax.axis_index("s")
    wid = c * NS + s
    gbase = wid * QG
    for j in range(B // 16):
        ones_v[pl.ds(j * 16, 16)] = jnp.ones((16,), jnp.float32)

    def zb(i, carry):
        zbuf[pl.ds(i * 16, 16)] = jnp.zeros((16,), jnp.float32)
        return carry

    lax.fori_loop(0, RPS // 16, zb, 0)
    zbuf[pl.ds(RPS - 16, 16)] = jnp.zeros((16,), jnp.float32)
    pltpu.sync_copy(zbuf, dacc.at[pl.ds(s * RPS, RPS)])

    def issue_idx(t):
        tb = lax.rem(t, 4)
        pltpu.async_copy(dstp8_hbm.at[gbase + t], didx8.at[tb], isem.at[tb])

    def wait_idx(t):
        tb = lax.rem(t, 4)
        pltpu.make_async_copy(dstp8_hbm.at[gbase], didx8.at[tb], isem.at[tb]).wait()

    def start_scatter(qb, u):
        pltpu.async_copy(ones_v, dacc.at[didx8.at[qb, u]],
                         ssem.at[u % 2], add=True)

    def wait_scatter(b):
        pltpu.make_async_copy(ones_v, dacc.at[didx8.at[0, 0]],
                              ssem.at[b]).wait()

    issue_idx(0)
    issue_idx(1)
    issue_idx(2)
    plsc.subcore_barrier()
    wait_idx(0)

    def body(q, carry):
        qb = lax.rem(q, 4)
        for u in range(GSZ):
            i = q * GSZ + u
            if u == 2:
                @pl.when(q + 3 < QG)
                def _():
                    issue_idx(q + 3)
            if u == 5:
                @pl.when(q + 1 < QG)
                def _():
                    wait_idx(q + 1)

            @pl.when(i >= 2)
            def _():
                wait_scatter(u % 2)

            start_scatter(qb, u)
        return carry

    lax.fori_loop(0, QG, body, 0)
    wait_scatter(0)
    wait_scatter(1)
    plsc.subcore_barrier()
    pltpu.sync_copy(dacc.at[pl.ds(s * RPS, RPS)], zbuf)
    pltpu.sync_copy(zbuf, out_hbm.at[pl.ds(c * NACC + s * RPS, RPS)])


def _sc_deg(dstp8):
    return pl.kernel(
        _sc_deg_body,
        out_type=jax.ShapeDtypeStruct((NC * NACC,), jnp.float32),
        mesh=_mesh,
        scratch_types=[
            pltpu.VMEM((4, GSZ, B), jnp.int32),
            pltpu.VMEM((B,), jnp.float32),
            pltpu.VMEM((RPS,), jnp.float32),
            pltpu.VMEM_SHARED((NACC,), jnp.float32),
            pltpu.SemaphoreType.DMA((4,)),
            pltpu.SemaphoreType.DMA((2,)),
        ],
    )(dstp8)


def _sc_agg_body(tbl_hbm, srcp8_hbm, dstp8_hbm, zeros_hbm, out_hbm,
                 sidx8, didx8, rows, acc, isem, gsem, ssem):
    c = lax.axis_index("c")
    s = lax.axis_index("s")
    wid = c * NS + s
    gbase = wid * QG
    pltpu.sync_copy(zeros_hbm, acc.at[pl.ds(s * RPS, RPS)])

    def issue_idx(t):
        tb = lax.rem(t, 4)
        pltpu.async_copy(srcp8_hbm.at[gbase + t], sidx8.at[tb], isem.at[tb])
        pltpu.async_copy(dstp8_hbm.at[gbase + t], didx8.at[tb], isem.at[tb])

    def wait_idx(t):
        tb = lax.rem(t, 4)
        pltpu.make_async_copy(srcp8_hbm.at[gbase], sidx8.at[tb], isem.at[tb]).wait()
        pltpu.make_async_copy(dstp8_hbm.at[gbase], didx8.at[tb], isem.at[tb]).wait()

    def start_gather(qg, ug):
        b = ug % 2
        pltpu.async_copy(tbl_hbm.at[sidx8.at[qg, ug]], rows.at[b], gsem.at[b])

    def wait_gather(b):
        pltpu.make_async_copy(tbl_hbm.at[sidx8.at[0, 0]], rows.at[b],
                              gsem.at[b]).wait()

    def start_scatter(qb, u):
        pltpu.async_copy(rows.at[u % 2], acc.at[didx8.at[qb, u]],
                         ssem.at[u % 2], add=True)

    def wait_scatter(b):
        pltpu.make_async_copy(rows.at[b], acc.at[didx8.at[0, 0]],
                              ssem.at[b]).wait()

    issue_idx(0)
    issue_idx(1)
    issue_idx(2)
    plsc.subcore_barrier()
    wait_idx(0)
    start_gather(0, 0)

    def body(q, carry):
        qb = lax.rem(q, 4)
        for u in range(GSZ):
            i = q * GSZ + u
            if u == 2:
                @pl.when(q + 3 < QG)
                def _():
                    issue_idx(q + 3)
            if u == 5:
                @pl.when(q + 1 < QG)
                def _():
                    wait_idx(q + 1)
            wait_gather(u % 2)
            start_scatter(qb, u)

            @pl.when(i >= 1)
            def _():
                wait_scatter((u + 1) % 2)

            @pl.when(i + 1 < NBLK)
            def _():
                k = i + 1
                start_gather(lax.rem(k // GSZ, 4), (u + 1) % GSZ)
        return carry

    lax.fori_loop(0, QG, body, 0)
    wait_scatter((NBLK - 1) % 2)
    plsc.subcore_barrier()
    pltpu.sync_copy(acc.at[pl.ds(s * RPS, RPS)],
                    out_hbm.at[pl.ds(c * NACC + s * RPS, RPS)])


def _sc_agg(tbl, srcp8, dstp8, zerosD):
    return pl.kernel(
        _sc_agg_body,
        out_type=jax.ShapeDtypeStruct((NC * NACC, D), jnp.float32),
        mesh=_mesh,
        scratch_types=[
            pltpu.VMEM((4, GSZ, B), jnp.int32),
            pltpu.VMEM((4, GSZ, B), jnp.int32),
            pltpu.VMEM((2, B, D), jnp.float32),
            pltpu.VMEM_SHARED((NACC, D), jnp.float32),
            pltpu.SemaphoreType.DMA((4,)),
            pltpu.SemaphoreType.DMA((2,)),
            pltpu.SemaphoreType.DMA((2,)),
        ],
    )(tbl, srcp8, dstp8, zerosD)


def _dinv_block(degp_ref):
    d = degp_ref[...]
    return lax.rsqrt(d[:, 0:1] + d[:, 1:2] + 1.0)


def _tc_scale_body(degp_ref, emb_ref, o_ref):
    i = pl.program_id(0)
    rowg = i * RB + lax.broadcasted_iota(jnp.int32, (RB, 1), 0)
    x = jnp.where(rowg < N, emb_ref[...], 0.0)
    o_ref[...] = x * _dinv_block(degp_ref)


def _tc_mm_body(degp_ref, s1a_ref, s1b_ref, x1_ref, w1_ref, b1_ref,
                w2_ref, o_ref):
    dinv = _dinv_block(degp_ref)
    s1 = (s1a_ref[...] + s1b_ref[...] + x1_ref[...]) * dinv
    h = jnp.dot(s1, w1_ref[...], preferred_element_type=jnp.float32) + b1_ref[...]
    v = jnp.dot(h, w2_ref[...], preferred_element_type=jnp.float32)
    o_ref[...] = v * dinv


def _tc_fin_body(degp_ref, s2a_ref, s2b_ref, vp_ref, b2_ref, g_ref,
                 be_ref, o_ref, st_ref):
    p = pl.program_id(0)
    i = pl.program_id(1)
    dinv = _dinv_block(degp_ref)
    pre = (s2a_ref[...] + s2b_ref[...] + vp_ref[...]) * dinv + b2_ref[...]

    @pl.when(p == 0)
    def _():
        @pl.when(i == 0)
        def _():
            st_ref[...] = jnp.zeros_like(st_ref)

        rowg = i * RB + lax.broadcasted_iota(jnp.int32, (RB, 1), 0)
        prem = jnp.where(rowg < N, pre, 0.0)
        sm = jnp.sum(prem, axis=0, keepdims=True)
        sq = jnp.sum(prem * prem, axis=0, keepdims=True)
        st_ref[...] += jnp.concatenate(
            [sm, sq, jnp.zeros((6, D), jnp.float32)], axis=0)
        o_ref[...] = pre

    @pl.when(p == 1)
    def _():
        mean = st_ref[0:1, :] / N
        ex2 = st_ref[1:2, :] / N
        var = ex2 - mean * mean
        o_ref[...] = (pre - mean) * lax.rsqrt(var + 1e-5) * g_ref[...] + be_ref[...]


_blk = lambda i: (i, 0)
_whole = lambda i: (0, 0)


def _row_spec():
    return pl.BlockSpec((RB, D), _blk)


def _tc_scale(degp, emb):
    return pl.pallas_call(
        _tc_scale_body,
        grid=(GRID,),
        in_specs=[
            pl.BlockSpec((RB, 2), _blk),
            _row_spec(),
        ],
        out_specs=_row_spec(),
        out_shape=jax.ShapeDtypeStruct((NACC, D), jnp.float32),
    )(degp, emb)


def _tc_mm(degp, s1, x1p, W1, b1, W2):
    return pl.pallas_call(
        _tc_mm_body,
        grid=(GRID,),
        in_specs=[
            pl.BlockSpec((RB, 2), _blk),
            pl.BlockSpec((RB, D), _blk),
            pl.BlockSpec((RB, D), lambda i: (i + NACC // RB, 0)),
            pl.BlockSpec((RB, D), _blk),
            pl.BlockSpec((D, 2 * D), _whole),
            pl.BlockSpec((1, 2 * D), _whole),
            pl.BlockSpec((2 * D, D), _whole),
        ],
        out_specs=pl.BlockSpec((RB, D), _blk),
        out_shape=jax.ShapeDtypeStruct((NACC, D), jnp.float32),
    )(degp, s1, s1, x1p, W1, b1, W2)


def _tc_fin(degp, s2, vpp, b2, gamma, beta):
    blk2 = lambda p, i: (i, 0)
    whole2 = lambda p, i: (0, 0)
    return pl.pallas_call(
        _tc_fin_body,
        grid=(2, GRID),
        in_specs=[
            pl.BlockSpec((RB, 2), blk2),
            pl.BlockSpec((RB, D), blk2),
            pl.BlockSpec((RB, D), lambda p, i: (i + NACC // RB, 0)),
            pl.BlockSpec((RB, D), blk2),
            pl.BlockSpec((1, D), whole2),
            pl.BlockSpec((1, D), whole2),
            pl.BlockSpec((1, D), whole2),
        ],
        out_specs=pl.BlockSpec((RB, D), blk2),
        out_shape=jax.ShapeDtypeStruct((N, D), jnp.float32),
        scratch_shapes=[pltpu.VMEM((8, D), jnp.float32)],
    )(degp, s2, s2, vpp, b2, gamma, beta)


def kernel(edge_index, emb, W1, b1, W2, b2, gamma, beta):
    src = edge_index[0].astype(jnp.int32)
    dst = edge_index[1].astype(jnp.int32)
    srcp = jnp.concatenate([src, _PAD_IDX])
    dstp = jnp.concatenate([dst, _PAD_IDX])
    srcp8 = srcp.reshape(NW * QG, GSZ, B)
    dstp8 = dstp.reshape(NW * QG, GSZ, B)

    zerosD = jnp.zeros((RPS, D), jnp.float32)

    degp_flat = _sc_deg(dstp8)
    degp = degp_flat.reshape(NC, NACC).T

    x1p = _tc_scale(degp, emb)

    s1 = _sc_agg(x1p, srcp8, dstp8, zerosD)

    vpp = _tc_mm(degp, s1, x1p, W1,
                 b1.reshape(1, 2 * D), W2)

    s2 = _sc_agg(vpp, srcp8, dstp8, zerosD)

    return _tc_fin(degp, s2, vpp, b2.reshape(1, D),
                   gamma.reshape(1, D), beta.reshape(1, D))
